# Initial kernel scaffold; baseline (speedup 1.0000x reference)
import jax
import jax.numpy as jnp
from jax.experimental import pallas as pl


def kernel(x0, lidar_img, rots, trans, intrins, post_rots, post_trans, shape_info):
    raise NotImplementedError("write your pallas kernel here")


# speedup vs baseline: 16.8911x; 16.8911x over previous
import functools

import jax
import jax.numpy as jnp
import numpy as np
from jax import lax
from jax.experimental import pallas as pl
from jax.experimental.pallas import tpu as pltpu
from jax.experimental.pallas import tpu_sc as plsc

_B, _N, _IMH, _IMW = 4, 6, 256, 704
_CAMC = 64
_DS = 16
_FH, _FW = _IMH // _DS, _IMW // _DS
_BN = _B * _N
_PPB = _N * _FH * _FW
_GRID = 256 * 256
_SENT = _GRID

_DXn = np.array([0.4, 0.4, 20.0], dtype=np.float32)
_BXn = np.array([-51.2 + 0.2, -51.2 + 0.2, -10.0 + 10.0], dtype=np.float32)
_OFF = _BXn - _DXn / 2.0


def _frustum():
    xs = np.broadcast_to(
        np.linspace(0.0, _IMW - 1.0, _FW, dtype=np.float32).reshape(1, _FW), (_FH, _FW)
    )
    ys = np.broadcast_to(
        np.linspace(0.0, _IMH - 1.0, _FH, dtype=np.float32).reshape(_FH, 1), (_FH, _FW)
    )
    return np.ascontiguousarray(xs), np.ascontiguousarray(ys)


_FX, _FY = _frustum()


def _tc_pool_body(lid_ref, out_ref):
    x = lid_ref[0]
    m = x.reshape(_FH, _DS, _IMW).max(axis=1)
    mt = m.T
    out_ref[0] = mt.reshape(_FW, _DS, _FH).max(axis=1).T


def _tc_pool(lid):
    return pl.pallas_call(
        _tc_pool_body,
        grid=(_BN,),
        in_specs=[pl.BlockSpec((1, _IMH, _IMW), lambda i: (i, 0, 0))],
        out_specs=pl.BlockSpec((1, _FH, _FW), lambda i: (i, 0, 0)),
        out_shape=jax.ShapeDtypeStruct((_BN, _FH, _FW), jnp.float32),
    )(lid)


@functools.lru_cache(maxsize=None)
def _sc_scatter_kernel():
    mesh = plsc.VectorSubcoreMesh(core_axis_name="c", subcore_axis_name="s")
    return pl.kernel(
        _sc_scatter_body,
        out_type=jax.ShapeDtypeStruct((_B, _CAMC, _GRID), jnp.float32),
        mesh=mesh,
        scratch_types=[
            pltpu.VMEM((_GRID,), jnp.float32),
            pltpu.VMEM((_PPB,), jnp.int32),
            pltpu.VMEM((_PPB,), jnp.float32),
            pltpu.VMEM((_PPB,), jnp.float32),
            pltpu.SemaphoreType.DMA,
            pltpu.SemaphoreType.DMA,
        ],
        compiler_params=pltpu.CompilerParams(needs_layout_passes=False),
    )


def _sc_scatter_body(feat_hbm, vidx_hbm, out_hbm, acc, idxb, val0, val1, semv, semo):
    wid = lax.axis_index("s") * 2 + lax.axis_index("c")
    b = wid // 8
    c0 = (wid % 8) * 8

    pltpu.sync_copy(vidx_hbm.at[pl.ds(b * _PPB, _PPB)], idxb)

    zv = jnp.zeros((16,), jnp.float32)

    def zbody(i, carry):
        for u in range(8):
            acc[pl.ds((i * 8 + u) * 16, 16)] = zv
        return carry

    lax.fori_loop(0, _GRID // (8 * 16), zbody, 0)

    def start_fetch(c, buf):
        fhw = _FH * _FW
        return [
            pltpu.async_copy(
                feat_hbm.at[pl.ds(((b * _N + n) * _CAMC + c) * fhw, fhw)],
                buf.at[pl.ds(n * fhw, fhw)],
                semv,
            )
            for n in range(_N)
        ]

    def scatter_pass(valb, zero):
        def body(j, carry):
            for u in range(4):
                base = (j * 4 + u) * 16
                iv = idxb[pl.ds(base, 16)]
                mask = iv < _SENT
                if zero:
                    plsc.store_scatter(acc, [iv], zv, mask=mask)
                else:
                    vv = valb[pl.ds(base, 16)]
                    plsc.addupdate_scatter(acc, [iv], vv, mask=mask)
            return carry

        lax.fori_loop(0, _PPB // (4 * 16), body, 0)

    bufs = (val0, val1)
    hs = start_fetch(c0, val0)
    for k in range(8):
        c = c0 + k
        buf = bufs[k % 2]
        for h in hs:
            h.wait()
        if k < 7:
            hs = start_fetch(c + 1, bufs[(k + 1) % 2])
        scatter_pass(buf, zero=False)
        pltpu.async_copy(acc, out_hbm.at[b, c], semo).wait()
        if k < 7:
            scatter_pass(None, zero=True)


def kernel(x0, lidar_img, rots, trans, intrins, post_rots, post_trans, shape_info):
    lid = lidar_img.reshape(_BN, _IMH, _IMW)
    d = _tc_pool(lid).reshape(_B, _N, _FH, _FW)

    frust = jnp.stack(
        (jnp.asarray(_FX), jnp.asarray(_FY), jnp.ones((_FH, _FW), jnp.float32)), -1
    )
    points = frust[None, None] - post_trans[:, :, None, None, :]
    points = jnp.einsum("bnij,bnhwj->bnhwi", jnp.linalg.inv(post_rots), points)
    d = jnp.where(d == 0.0, 200.0, d)
    points = points * d[..., None]
    combine = jnp.einsum("bnij,bnjk->bnik", rots, jnp.linalg.inv(intrins))
    points = jnp.einsum("bnij,bnhwj->bnhwi", combine, points)
    geom = points + trans[:, :, None, None, :]

    gf = ((geom - jnp.asarray(_OFF)) / jnp.asarray(_DXn)).astype(jnp.int32)
    gfx, gfy, gfz = gf[..., 0], gf[..., 1], gf[..., 2]
    kept = (gfx >= 0) & (gfx < 256) & (gfy >= 0) & (gfy < 256) & (gfz >= 0) & (gfz < 1)
    vidx = jnp.where(kept, gfx * 256 + gfy, _SENT).reshape(_B * _PPB)

    feat = x0.reshape(_BN * _CAMC * _FH * _FW)
    out = _sc_scatter_kernel()(feat, vidx)
    return out.reshape(_B, _CAMC, 256, 256)


# speedup vs baseline: 20.6523x; 1.2227x over previous
import functools

import jax
import jax.numpy as jnp
import numpy as np
from jax import lax
from jax.experimental import pallas as pl
from jax.experimental.pallas import tpu as pltpu
from jax.experimental.pallas import tpu_sc as plsc

_B, _N, _IMH, _IMW = 4, 6, 256, 704
_CAMC = 64
_DS = 16
_FH, _FW = _IMH // _DS, _IMW // _DS
_BN = _B * _N
_PPB = _N * _FH * _FW
_GRID = 256 * 256
_SENT = _GRID

_DXn = np.array([0.4, 0.4, 20.0], dtype=np.float32)
_BXn = np.array([-51.2 + 0.2, -51.2 + 0.2, -10.0 + 10.0], dtype=np.float32)
_OFF = _BXn - _DXn / 2.0


def _frustum():
    xs = np.broadcast_to(
        np.linspace(0.0, _IMW - 1.0, _FW, dtype=np.float32).reshape(1, _FW), (_FH, _FW)
    )
    ys = np.broadcast_to(
        np.linspace(0.0, _IMH - 1.0, _FH, dtype=np.float32).reshape(_FH, 1), (_FH, _FW)
    )
    return np.ascontiguousarray(xs), np.ascontiguousarray(ys)


_FX, _FY = _frustum()


def _tc_pool_body(lid_ref, out_ref):
    x = lid_ref[0]
    m = x.reshape(_FH, _DS, _IMW).max(axis=1)
    mt = m.T
    out_ref[0] = mt.reshape(_FW, _DS, _FH).max(axis=1).T


def _tc_pool(lid):
    return pl.pallas_call(
        _tc_pool_body,
        grid=(_BN,),
        in_specs=[pl.BlockSpec((1, _IMH, _IMW), lambda i: (i, 0, 0))],
        out_specs=pl.BlockSpec((1, _FH, _FW), lambda i: (i, 0, 0)),
        out_shape=jax.ShapeDtypeStruct((_BN, _FH, _FW), jnp.float32),
    )(lid)


@functools.lru_cache(maxsize=None)
def _sc_scatter_kernel():
    mesh = plsc.VectorSubcoreMesh(core_axis_name="c", subcore_axis_name="s")
    return pl.kernel(
        _sc_scatter_body,
        out_type=jax.ShapeDtypeStruct((_B, _CAMC, 256, 256), jnp.float32),
        mesh=mesh,
        scratch_types=[
            pltpu.VMEM((256, 256), jnp.float32),
            pltpu.VMEM((_PPB,), jnp.int32),
            pltpu.VMEM((_PPB,), jnp.float32),
            pltpu.VMEM((_PPB,), jnp.float32),
            pltpu.SemaphoreType.DMA,
            pltpu.SemaphoreType.DMA,
        ],
        compiler_params=pltpu.CompilerParams(needs_layout_passes=False),
    )


def _sc_scatter_body(feat_hbm, vidx_hbm, out_hbm, acc, idxb, val0, val1, semv, semo):
    wid = lax.axis_index("s") * 2 + lax.axis_index("c")
    b = wid // 8
    c0 = (wid % 8) * 8

    pltpu.sync_copy(vidx_hbm.at[pl.ds(b * _PPB, _PPB)], idxb)

    zv = jnp.zeros((16,), jnp.float32)

    def zbody(i, carry):
        for u in range(16):
            acc[i, pl.ds(u * 16, 16)] = zv
        return carry

    lax.fori_loop(0, 256, zbody, 0)

    def start_fetch(c, buf):
        fhw = _FH * _FW
        return [
            pltpu.async_copy(
                feat_hbm.at[pl.ds(((b * _N + n) * _CAMC + c) * fhw, fhw)],
                buf.at[pl.ds(n * fhw, fhw)],
                semv,
            )
            for n in range(_N)
        ]

    def scatter_pass(valb, zero):
        def body(j, carry):
            for u in range(4):
                base = (j * 4 + u) * 16
                iv = idxb[pl.ds(base, 16)]
                mask = iv < _SENT
                ix = lax.shift_right_logical(iv, 8)
                iy = lax.bitwise_and(iv, 255)
                if zero:
                    plsc.store_scatter(acc, [ix, iy], zv, mask=mask)
                else:
                    vv = valb[pl.ds(base, 16)]
                    plsc.addupdate_scatter(acc, [ix, iy], vv, mask=mask)
            return carry

        lax.fori_loop(0, _PPB // (4 * 16), body, 0)

    bufs = (val0, val1)
    hs = start_fetch(c0, val0)
    for k in range(8):
        c = c0 + k
        buf = bufs[k % 2]
        for h in hs:
            h.wait()
        if k < 7:
            hs = start_fetch(c + 1, bufs[(k + 1) % 2])
        scatter_pass(buf, zero=False)
        pltpu.async_copy(acc, out_hbm.at[b, c], semo).wait()
        if k < 7:
            scatter_pass(None, zero=True)


def kernel(x0, lidar_img, rots, trans, intrins, post_rots, post_trans, shape_info):
    lid = lidar_img.reshape(_BN, _IMH, _IMW)
    d = _tc_pool(lid).reshape(_B, _N, _FH, _FW)

    frust = jnp.stack(
        (jnp.asarray(_FX), jnp.asarray(_FY), jnp.ones((_FH, _FW), jnp.float32)), -1
    )
    points = frust[None, None] - post_trans[:, :, None, None, :]
    points = jnp.einsum("bnij,bnhwj->bnhwi", jnp.linalg.inv(post_rots), points)
    d = jnp.where(d == 0.0, 200.0, d)
    points = points * d[..., None]
    combine = jnp.einsum("bnij,bnjk->bnik", rots, jnp.linalg.inv(intrins))
    points = jnp.einsum("bnij,bnhwj->bnhwi", combine, points)
    geom = points + trans[:, :, None, None, :]

    gf = ((geom - jnp.asarray(_OFF)) / jnp.asarray(_DXn)).astype(jnp.int32)
    gfx, gfy, gfz = gf[..., 0], gf[..., 1], gf[..., 2]
    kept = (gfx >= 0) & (gfx < 256) & (gfy >= 0) & (gfy < 256) & (gfz >= 0) & (gfz < 1)
    vidx = jnp.where(kept, gfx * 256 + gfy, _SENT).reshape(_B * _PPB)

    feat = x0.reshape(_BN * _CAMC * _FH * _FW)
    out = _sc_scatter_kernel()(feat, vidx)
    return out.reshape(_B, _CAMC, 256, 256)
